# Initial kernel scaffold; baseline (speedup 1.0000x reference)
import jax
import jax.numpy as jnp
from jax.experimental import pallas as pl


def kernel(mem, dst_ids, msgs, query_ids):
    raise NotImplementedError("write your pallas kernel here")


# speedup vs baseline: 3.0263x; 3.0263x over previous
import functools

import jax
import jax.numpy as jnp
from jax import lax
from jax.experimental import pallas as pl
from jax.experimental.pallas import tpu as pltpu
from jax.experimental.pallas import tpu_sc as plsc

M = 1000000
B = 16384
D = 64
L = 16
NC = 2
NS = 16
NW = NC * NS
R = 65536
MP = NS * R
QT = B // NW
GC = 128


def _body(mem_hbm, dst_hbm, msgs_hbm, q_hbm, out_hbm, win_hbm):
    c = lax.axis_index("c")
    s = lax.axis_index("s")
    wid = c * NS + s

    def phase_build(dst_v, win_v):
        pltpu.sync_copy(dst_hbm, dst_v)

        neg1 = jnp.full((L,), -1, dtype=jnp.int32)

        def memset_body(j, _):
            for k in range(8):
                win_v[pl.ds((j * 8 + k) * L, L)] = neg1
            return 0

        lax.fori_loop(0, R // (8 * L), memset_body, 0)

        lanes = lax.iota(jnp.int32, L)

        def scan_body(j, _):
            v = dst_v[pl.ds(j * L, L)]
            pos = lanes + j * L
            m = (v >> 16) == s
            local = jnp.where(m, v & 0xFFFF, 0)
            plsc.store_scatter(win_v, [local], pos, mask=m)
            g = plsc.load_gather(win_v, [local])
            need = m & (g < pos)

            def conflict_cond(carry):
                return jnp.any(carry)

            def conflict_body(carry):
                plsc.store_scatter(win_v, [local], pos, mask=carry)
                g2 = plsc.load_gather(win_v, [local])
                return m & (g2 < pos)

            lax.while_loop(conflict_cond, conflict_body, need)
            return 0

        lax.fori_loop(0, B // L, scan_body, 0)

        pltpu.sync_copy(win_v, win_hbm.at[pl.ds(s * R, R)])

    pl.run_scoped(
        phase_build,
        pltpu.VMEM((B,), jnp.int32),
        pltpu.VMEM((R,), jnp.int32),
    )

    plsc.subcore_barrier()

    def phase_query(q_v, wq_v, mi_v, rows_a, rows_b, sem):
        base_q = wid * QT
        pltpu.sync_copy(q_hbm.at[pl.ds(base_q, QT)], q_v)

        for t in range(QT // GC):
            sl = pl.ds(t * GC, GC)
            pltpu.async_copy(win_hbm.at[q_v.at[sl]], wq_v.at[sl], sem).wait()

        def safeidx_body(j, _):
            w = wq_v[pl.ds(j * L, L)]
            mi_v[pl.ds(j * L, L)] = jnp.where(w >= 0, w, 0)
            return 0

        lax.fori_loop(0, QT // L, safeidx_body, 0)

        gathers = []
        for t in range(QT // GC):
            sl = pl.ds(t * GC, GC)
            gathers.append(
                pltpu.async_copy(mem_hbm.at[q_v.at[sl]], rows_a.at[sl], sem))
            gathers.append(
                pltpu.async_copy(msgs_hbm.at[mi_v.at[sl]], rows_b.at[sl], sem))
        for g in gathers:
            g.wait()

        def select_body(j, _):
            wvec = wq_v[pl.ds(j * L, L)]
            for k in range(L):
                valid = wvec[k] >= 0
                i = j * L + k
                for dv in range(D // L):
                    a = rows_b[i, pl.ds(dv * L, L)]
                    b = rows_a[i, pl.ds(dv * L, L)]
                    rows_a[i, pl.ds(dv * L, L)] = jnp.where(valid, a, b)
            return 0

        lax.fori_loop(0, QT // L, select_body, 0)

        pltpu.sync_copy(rows_a, out_hbm.at[pl.ds(base_q, QT)])

    pl.run_scoped(
        phase_query,
        pltpu.VMEM((QT,), jnp.int32),
        pltpu.VMEM((QT,), jnp.int32),
        pltpu.VMEM((QT,), jnp.int32),
        pltpu.VMEM((QT, D), jnp.float32),
        pltpu.VMEM((QT, D), jnp.float32),
        pltpu.SemaphoreType.DMA,
    )


@jax.jit
def kernel(mem, dst_ids, msgs, query_ids):
    mesh = plsc.VectorSubcoreMesh(core_axis_name="c", subcore_axis_name="s")
    out, _ = pl.kernel(
        _body,
        out_type=(
            jax.ShapeDtypeStruct((B, D), jnp.float32),
            jax.ShapeDtypeStruct((MP,), jnp.int32),
        ),
        mesh=mesh,
        compiler_params=pltpu.CompilerParams(
            needs_layout_passes=False, use_tc_tiling_on_sc=False),
    )(mem, dst_ids, msgs, query_ids)
    return out
